# baseline (speedup 1.0000x reference)
import functools

import jax
import jax.numpy as jnp
from jax import lax
from jax.experimental import pallas as pl
from jax.experimental.pallas import tpu as pltpu
from jax.experimental.pallas import tpu_sc as plsc

HD = 192
ED = 64
NBATCH = 1024
PP = 2
TB = PP * NBATCH
CH = 128


def _sc_gather6(tables, idxs, T):
    info = plsc.get_sparse_core_info()
    NC, NS = info.num_cores, info.num_subcores
    NW = NC * NS
    per_w = T // NW
    n_ch = per_w // CH
    mesh = plsc.VectorSubcoreMesh(core_axis_name="c", subcore_axis_name="s")

    @functools.partial(
        pl.kernel,
        mesh=mesh,
        compiler_params=pltpu.CompilerParams(use_tc_tiling_on_sc=False),
        out_type=[jax.ShapeDtypeStruct((T, ED), jnp.float32) for _ in range(6)],
        scratch_types=(
            [pltpu.VMEM((CH,), jnp.int32) for _ in range(6)]
            + [pltpu.VMEM((CH, ED), jnp.float32) for _ in range(6)]
            + [pltpu.SemaphoreType.DMA, pltpu.SemaphoreType.DMA,
               pltpu.SemaphoreType.DMA]
        ),
    )
    def k(t0, t1, t2, t3, t4, t5, i0, i1, i2, i3, i4, i5,
          o0, o1, o2, o3, o4, o5, v0, v1, v2, v3, v4, v5,
          r0, r1, r2, r3, r4, r5, sem_i, sem_g, sem_w):
        tabs = (t0, t1, t2, t3, t4, t5)
        idxv = (v0, v1, v2, v3, v4, v5)
        rows = (r0, r1, r2, r3, r4, r5)
        outs = (o0, o1, o2, o3, o4, o5)
        wid = lax.axis_index("s") * NC + lax.axis_index("c")
        base = wid * per_w

        def body(c, carry):
            off = base + c * CH
            loads = [
                pltpu.async_copy(idx.at[pl.ds(off, CH)], v, sem_i)
                for idx, v in zip((i0, i1, i2, i3, i4, i5), idxv)
            ]
            for cp in loads:
                cp.wait()
            gathers = [
                pltpu.async_copy(tabs[f].at[idxv[f]], rows[f], sem_g)
                for f in range(6)
            ]
            for cp in gathers:
                cp.wait()
            writes = [
                pltpu.async_copy(rows[f], outs[f].at[pl.ds(off, CH)], sem_w)
                for f in range(6)
            ]
            for cp in writes:
                cp.wait()
            return carry

        lax.fori_loop(0, n_ch, body, 0)

    return k(*tables, *idxs)


def _tc_body(el_r, td_r, gn, ga, gt, gg, gu, gi,
             wci, wca, wct, wcg, wcu, wcit, wea, wet, weg,
             wcel, wctd, weel, wetd, bc, be, eo, xo):
    f32 = jnp.float32
    fields = (gn, ga, gt, gg, gu, gi)
    wx = (wci, wca, wct, wcg, wcu, wcit)
    we = (wea, wet, weg)
    for p in range(PP):
        sl = slice(p * ED, (p + 1) * ED)
        x = jnp.dot(fields[0][...][:, sl], wx[0][...],
                    preferred_element_type=f32)
        for f in range(1, 6):
            x += jnp.dot(fields[f][...][:, sl], wx[f][...],
                         preferred_element_type=f32)
        e = jnp.dot(fields[1][...][:, sl], we[0][...],
                    preferred_element_type=f32)
        e += jnp.dot(fields[2][...][:, sl], we[1][...],
                     preferred_element_type=f32)
        e += jnp.dot(fields[3][...][:, sl], we[2][...],
                     preferred_element_type=f32)
        el = el_r[0, p, :][:, None]
        td = td_r[0, p, :][:, None]
        x += el * wcel[...]
        x += td * wctd[...]
        x += bc[...]
        e += el * weel[...]
        e += td * wetd[...]
        e += be[...]
        xo[p] = x.T
        eo[p] = e.T


def _tc_project(el3, td3, gs, wblocks, T, S):
    NB = T // TB
    pair_spec = pl.BlockSpec((TB // 2, 2 * ED), lambda i: (i, 0))
    tok_spec = pl.BlockSpec((1, PP, NBATCH), lambda i: (i, 0, 0))
    full = lambda s: pl.BlockSpec(s, lambda i: (0, 0))
    in_specs = (
        [tok_spec, tok_spec]
        + [pair_spec] * 6
        + [full(w.shape) for w in wblocks]
    )
    out_specs = [pl.BlockSpec((PP, HD, NBATCH), lambda i: (i, 0, 0))] * 2
    out_shape = [jax.ShapeDtypeStruct((S, HD, NBATCH), jnp.float32)] * 2
    return pl.pallas_call(
        _tc_body,
        grid=(NB,),
        in_specs=in_specs,
        out_specs=out_specs,
        out_shape=out_shape,
    )(el3, td3, *gs, *wblocks)


def kernel(interaction, user_idx, item_idx, assessmentItemID, testId, KnowledgeTag,
           elapsed, time_diff, user_emb, item_emb, emb_interaction, emb_assess,
           emb_test, emb_tag, W_comb, b_comb, W_enc, b_enc):
    B, S = interaction.shape
    T = B * S
    NB = T // TB

    i32 = jnp.int32

    def _stream(z):
        return (z.T.reshape(NB, PP, NBATCH).transpose(0, 2, 1)
                .reshape(-1).astype(i32))

    idx_n = _stream(interaction)
    idx_a = _stream(assessmentItemID)
    idx_t = _stream(testId)
    idx_g = _stream(KnowledgeTag)
    idx_u = _stream(user_idx)
    idx_i = _stream(item_idx)

    emb_inter8 = jnp.concatenate(
        [emb_interaction, jnp.zeros((5, ED), jnp.float32)], axis=0)
    gs = _sc_gather6(
        (emb_inter8, emb_assess, emb_test, emb_tag, user_emb, item_emb),
        (idx_n, idx_a, idx_t, idx_g, idx_u, idx_i), T)
    gs = [g.reshape(T // 2, 2 * ED) for g in gs]

    el3 = elapsed.T.reshape(NB, PP, NBATCH)
    td3 = time_diff.T.reshape(NB, PP, NBATCH)

    wblocks = (
        W_comb[0:64],
        W_comb[64:128],
        W_comb[128:192],
        W_comb[192:256],
        W_comb[258:322],
        W_comb[322:386],
        W_enc[0:64],
        W_enc[64:128],
        W_enc[128:192],
        W_comb[256:257],
        W_comb[257:258],
        W_enc[192:193],
        W_enc[193:194],
        b_comb.reshape(1, HD),
        b_enc.reshape(1, HD),
    )
    enc_x, x = _tc_project(el3, td3, gs, wblocks, T, S)
    return (jnp.transpose(enc_x, (2, 0, 1)), jnp.transpose(x, (2, 0, 1)))


# speedup vs baseline: 4.4039x; 4.4039x over previous
import functools

import jax
import jax.numpy as jnp
from jax import lax
from jax.experimental import pallas as pl
from jax.experimental.pallas import tpu as pltpu
from jax.experimental.pallas import tpu_sc as plsc

HD = 192
ED = 64
NBATCH = 1024
PP = 2
TB = PP * NBATCH
CH = 128


def _sc_gather5(tables, idx5, T):
    info = plsc.get_sparse_core_info()
    NC, NS = info.num_cores, info.num_subcores
    NW = NC * NS
    per_w = T // NW
    n_ch = per_w // CH
    mesh = plsc.VectorSubcoreMesh(core_axis_name="c", subcore_axis_name="s")

    @functools.partial(
        pl.kernel,
        mesh=mesh,
        compiler_params=pltpu.CompilerParams(use_tc_tiling_on_sc=False),
        out_type=[jax.ShapeDtypeStruct((T, ED), jnp.float32) for _ in range(5)],
        scratch_types=(
            [pltpu.VMEM((CH,), jnp.int32) for _ in range(5)]
            + [pltpu.VMEM((CH, ED), jnp.float32) for _ in range(5)]
            + [pltpu.SemaphoreType.DMA, pltpu.SemaphoreType.DMA,
               pltpu.SemaphoreType.DMA]
        ),
    )
    def k(t0, t1, t2, t3, t4, i5,
          o0, o1, o2, o3, o4, v0, v1, v2, v3, v4,
          r0, r1, r2, r3, r4, sem_i, sem_g, sem_w):
        tabs = (t0, t1, t2, t3, t4)
        idxv = (v0, v1, v2, v3, v4)
        rows = (r0, r1, r2, r3, r4)
        outs = (o0, o1, o2, o3, o4)
        wid = lax.axis_index("s") * NC + lax.axis_index("c")
        base = wid * per_w

        def body(c, carry):
            off = base + c * CH
            loads = [
                pltpu.async_copy(i5.at[f, pl.ds(off, CH)], idxv[f], sem_i)
                for f in range(5)
            ]
            for cp in loads:
                cp.wait()
            gathers = [
                pltpu.async_copy(tabs[f].at[idxv[f]], rows[f], sem_g)
                for f in range(5)
            ]
            for cp in gathers:
                cp.wait()
            writes = [
                pltpu.async_copy(rows[f], outs[f].at[pl.ds(off, CH)], sem_w)
                for f in range(5)
            ]
            for cp in writes:
                cp.wait()
            return carry

        lax.fori_loop(0, n_ch, body, 0)

    return k(*tables, idx5)


def _tc_body(inter_r, el_r, td_r, ga, gt, gg, gu, gi,
             eint, wci, wca, wct, wcg, wcu, wcit, wea, wet, weg,
             wcel, wctd, weel, wetd, bc, be, eo, xo):
    f32 = jnp.float32
    fields = (ga, gt, gg, gu, gi)
    wx = (wca, wct, wcg, wcu, wcit)
    we = (wea, wet, weg)
    m3 = jnp.dot(eint[...], wci[...], preferred_element_type=f32)
    inter = inter_r[0, 0, :]
    elv = el_r[0, 0, :]
    tdv = td_r[0, 0, :]
    for p in range(PP):
        sl = slice(p * ED, (p + 1) * ED)
        x = jnp.dot(fields[0][...][:, sl], wx[0][...],
                    preferred_element_type=f32)
        for f in range(1, 5):
            x += jnp.dot(fields[f][...][:, sl], wx[f][...],
                         preferred_element_type=f32)
        e = jnp.dot(fields[0][...][:, sl], we[0][...],
                    preferred_element_type=f32)
        e += jnp.dot(fields[1][...][:, sl], we[1][...],
                     preferred_element_type=f32)
        e += jnp.dot(fields[2][...][:, sl], we[2][...],
                     preferred_element_type=f32)
        ii = inter[p * NBATCH:(p + 1) * NBATCH][:, None]
        x += jnp.where(ii == 0, 1.0, 0.0) * m3[0:1, :]
        x += jnp.where(ii == 1, 1.0, 0.0) * m3[1:2, :]
        x += jnp.where(ii == 2, 1.0, 0.0) * m3[2:3, :]
        el = elv[p * NBATCH:(p + 1) * NBATCH][:, None]
        td = tdv[p * NBATCH:(p + 1) * NBATCH][:, None]
        x += el * wcel[...]
        x += td * wctd[...]
        x += bc[...]
        e += el * weel[...]
        e += td * wetd[...]
        e += be[...]
        xo[p] = x.T
        eo[p] = e.T


def _tc_project(inter3, el3, td3, gs, eint, wblocks, T, S):
    NB = T // TB
    pair_spec = pl.BlockSpec((TB // 2, 2 * ED), lambda i: (i, 0))
    tok_spec = pl.BlockSpec((1, 1, TB), lambda i: (i, 0, 0))
    full = lambda s: pl.BlockSpec(s, lambda i: (0, 0))
    in_specs = (
        [tok_spec, tok_spec, tok_spec]
        + [pair_spec] * 5
        + [full(w.shape) for w in ([eint] + list(wblocks))]
    )
    out_specs = [pl.BlockSpec((PP, HD, NBATCH), lambda i: (i, 0, 0))] * 2
    out_shape = [jax.ShapeDtypeStruct((S, HD, NBATCH), jnp.float32)] * 2
    return pl.pallas_call(
        _tc_body,
        grid=(NB,),
        in_specs=in_specs,
        out_specs=out_specs,
        out_shape=out_shape,
    )(inter3, el3, td3, *gs, eint, *wblocks)


def kernel(interaction, user_idx, item_idx, assessmentItemID, testId, KnowledgeTag,
           elapsed, time_diff, user_emb, item_emb, emb_interaction, emb_assess,
           emb_test, emb_tag, W_comb, b_comb, W_enc, b_enc):
    B, S = interaction.shape
    T = B * S
    NB = T // TB

    idx5 = jnp.stack([assessmentItemID, testId, KnowledgeTag,
                      user_idx, item_idx]).astype(jnp.int32)
    idx5 = (idx5.transpose(0, 2, 1).reshape(5, NB, PP, NBATCH)
            .transpose(0, 1, 3, 2).reshape(5, T))

    gs = _sc_gather5(
        (emb_assess, emb_test, emb_tag, user_emb, item_emb), idx5, T)
    gs = [g.reshape(T // 2, 2 * ED) for g in gs]

    inter3 = interaction.T.reshape(NB, 1, TB).astype(jnp.int32)
    el3 = elapsed.T.reshape(NB, 1, TB)
    td3 = time_diff.T.reshape(NB, 1, TB)

    wblocks = (
        W_comb[0:64],
        W_comb[64:128],
        W_comb[128:192],
        W_comb[192:256],
        W_comb[258:322],
        W_comb[322:386],
        W_enc[0:64],
        W_enc[64:128],
        W_enc[128:192],
        W_comb[256:257],
        W_comb[257:258],
        W_enc[192:193],
        W_enc[193:194],
        b_comb.reshape(1, HD),
        b_enc.reshape(1, HD),
    )
    enc_x, x = _tc_project(inter3, el3, td3, gs, emb_interaction, wblocks, T, S)
    return (jnp.transpose(enc_x, (2, 0, 1)), jnp.transpose(x, (2, 0, 1)))


# speedup vs baseline: 4.6675x; 1.0599x over previous
import functools

import jax
import jax.numpy as jnp
from jax import lax
from jax.experimental import pallas as pl
from jax.experimental.pallas import tpu as pltpu
from jax.experimental.pallas import tpu_sc as plsc

HD = 192
ED = 64
NBATCH = 1024
PP = 2
TB = PP * NBATCH
CH = 128


def _sc_gather5(tables, idx5, T):
    info = plsc.get_sparse_core_info()
    NC, NS = info.num_cores, info.num_subcores
    NW = NC * NS
    per_w = T // NW
    n_ch = per_w // CH
    mesh = plsc.VectorSubcoreMesh(core_axis_name="c", subcore_axis_name="s")

    @functools.partial(
        pl.kernel,
        mesh=mesh,
        compiler_params=pltpu.CompilerParams(use_tc_tiling_on_sc=False),
        out_type=[jax.ShapeDtypeStruct((T, ED), jnp.float32) for _ in range(5)],
        scratch_types=(
            [pltpu.VMEM((2, CH), jnp.int32) for _ in range(5)]
            + [pltpu.VMEM((2, CH, ED), jnp.float32) for _ in range(5)]
            + [pltpu.SemaphoreType.DMA, pltpu.SemaphoreType.DMA,
               pltpu.SemaphoreType.DMA]
        ),
    )
    def k(t0, t1, t2, t3, t4, i5,
          o0, o1, o2, o3, o4, v0, v1, v2, v3, v4,
          r0, r1, r2, r3, r4, sem_i, sem_g, sem_w):
        tabs = (t0, t1, t2, t3, t4)
        idxv = (v0, v1, v2, v3, v4)
        rows = (r0, r1, r2, r3, r4)
        outs = (o0, o1, o2, o3, o4)
        wid = lax.axis_index("s") * NC + lax.axis_index("c")
        base = wid * per_w

        def idx_cps(c, b, mk):
            return [mk(i5.at[f, pl.ds(base + c * CH, CH)], idxv[f].at[b],
                       sem_i) for f in range(5)]

        def gather_cps(b, mk):
            return [mk(tabs[f].at[idxv[f].at[b]], rows[f].at[b], sem_g)
                    for f in range(5)]

        def write_cps(c, b, mk):
            return [mk(rows[f].at[b], outs[f].at[pl.ds(base + c * CH, CH)],
                       sem_w) for f in range(5)]

        issue, mk = pltpu.async_copy, pltpu.make_async_copy

        idx_cps(0, 0, issue)
        for cp in idx_cps(0, 0, mk):
            cp.wait()
        gather_cps(0, issue)
        idx_cps(1, 1, issue)

        def half(c, b):
            for cp in gather_cps(b, mk):
                cp.wait()

            @pl.when(c >= 1)
            def _():
                for cp in write_cps(c - 1, 1 - b, mk):
                    cp.wait()

            @pl.when(c + 1 < n_ch)
            def _():
                for cp in idx_cps(c + 1, 1 - b, mk):
                    cp.wait()
                gather_cps(1 - b, issue)

            @pl.when(c + 2 < n_ch)
            def _():
                idx_cps(c + 2, b, issue)

            write_cps(c, b, issue)

        def body(c2, carry):
            half(2 * c2, 0)
            half(2 * c2 + 1, 1)
            return carry

        lax.fori_loop(0, n_ch // 2, body, 0)
        for cp in write_cps(n_ch - 1, 1, mk):
            cp.wait()

    return k(*tables, idx5)


def _tc_body(inter_r, el_r, td_r, ga, gt, gg, gu, gi,
             eint, wci, wca, wct, wcg, wcu, wcit, wea, wet, weg,
             wcel, wctd, weel, wetd, bc, be, eo, xo):
    f32 = jnp.float32
    fields = (ga, gt, gg, gu, gi)
    wx = (wca, wct, wcg, wcu, wcit)
    we = (wea, wet, weg)
    m3 = jnp.dot(eint[...], wci[...], preferred_element_type=f32)
    inter = inter_r[0, 0, :]
    elv = el_r[0, 0, :]
    tdv = td_r[0, 0, :]
    for p in range(PP):
        sl = slice(p * ED, (p + 1) * ED)
        x = jnp.dot(fields[0][...][:, sl], wx[0][...],
                    preferred_element_type=f32)
        for f in range(1, 5):
            x += jnp.dot(fields[f][...][:, sl], wx[f][...],
                         preferred_element_type=f32)
        e = jnp.dot(fields[0][...][:, sl], we[0][...],
                    preferred_element_type=f32)
        e += jnp.dot(fields[1][...][:, sl], we[1][...],
                     preferred_element_type=f32)
        e += jnp.dot(fields[2][...][:, sl], we[2][...],
                     preferred_element_type=f32)
        ii = inter[p * NBATCH:(p + 1) * NBATCH][:, None]
        x += jnp.where(ii == 0, 1.0, 0.0) * m3[0:1, :]
        x += jnp.where(ii == 1, 1.0, 0.0) * m3[1:2, :]
        x += jnp.where(ii == 2, 1.0, 0.0) * m3[2:3, :]
        el = elv[p * NBATCH:(p + 1) * NBATCH][:, None]
        td = tdv[p * NBATCH:(p + 1) * NBATCH][:, None]
        x += el * wcel[...]
        x += td * wctd[...]
        x += bc[...]
        e += el * weel[...]
        e += td * wetd[...]
        e += be[...]
        xo[p] = x.T
        eo[p] = e.T


def _tc_project(inter3, el3, td3, gs, eint, wblocks, T, S):
    NB = T // TB
    pair_spec = pl.BlockSpec((TB // 2, 2 * ED), lambda i: (i, 0))
    tok_spec = pl.BlockSpec((1, 1, TB), lambda i: (i, 0, 0))
    full = lambda s: pl.BlockSpec(s, lambda i: (0, 0))
    in_specs = (
        [tok_spec, tok_spec, tok_spec]
        + [pair_spec] * 5
        + [full(w.shape) for w in ([eint] + list(wblocks))]
    )
    out_specs = [pl.BlockSpec((PP, HD, NBATCH), lambda i: (i, 0, 0))] * 2
    out_shape = [jax.ShapeDtypeStruct((S, HD, NBATCH), jnp.float32)] * 2
    return pl.pallas_call(
        _tc_body,
        grid=(NB,),
        in_specs=in_specs,
        out_specs=out_specs,
        out_shape=out_shape,
    )(inter3, el3, td3, *gs, eint, *wblocks)


def kernel(interaction, user_idx, item_idx, assessmentItemID, testId, KnowledgeTag,
           elapsed, time_diff, user_emb, item_emb, emb_interaction, emb_assess,
           emb_test, emb_tag, W_comb, b_comb, W_enc, b_enc):
    B, S = interaction.shape
    T = B * S
    NB = T // TB

    idx5 = jnp.stack([assessmentItemID, testId, KnowledgeTag,
                      user_idx, item_idx]).astype(jnp.int32)
    idx5 = (idx5.transpose(0, 2, 1).reshape(5, NB, PP, NBATCH)
            .transpose(0, 1, 3, 2).reshape(5, T))

    gs = _sc_gather5(
        (emb_assess, emb_test, emb_tag, user_emb, item_emb), idx5, T)
    gs = [g.reshape(T // 2, 2 * ED) for g in gs]

    inter3 = interaction.T.reshape(NB, 1, TB).astype(jnp.int32)
    el3 = elapsed.T.reshape(NB, 1, TB)
    td3 = time_diff.T.reshape(NB, 1, TB)

    wblocks = (
        W_comb[0:64],
        W_comb[64:128],
        W_comb[128:192],
        W_comb[192:256],
        W_comb[258:322],
        W_comb[322:386],
        W_enc[0:64],
        W_enc[64:128],
        W_enc[128:192],
        W_comb[256:257],
        W_comb[257:258],
        W_enc[192:193],
        W_enc[193:194],
        b_comb.reshape(1, HD),
        b_enc.reshape(1, HD),
    )
    enc_x, x = _tc_project(inter3, el3, td3, gs, emb_interaction, wblocks, T, S)
    return (jnp.transpose(enc_x, (2, 0, 1)), jnp.transpose(x, (2, 0, 1)))


# speedup vs baseline: 4.7738x; 1.0228x over previous
import functools

import jax
import jax.numpy as jnp
from jax import lax
from jax.experimental import pallas as pl
from jax.experimental.pallas import tpu as pltpu
from jax.experimental.pallas import tpu_sc as plsc

HD = 192
ED = 64
NBATCH = 1024
PP = 2
TB = PP * NBATCH
CH = 128


def _sc_gather5(tables, idx5, T):
    info = plsc.get_sparse_core_info()
    NC, NS = info.num_cores, info.num_subcores
    NW = NC * NS
    per_w = T // NW
    n_ch = per_w // CH
    mesh = plsc.VectorSubcoreMesh(core_axis_name="c", subcore_axis_name="s")

    @functools.partial(
        pl.kernel,
        mesh=mesh,
        compiler_params=pltpu.CompilerParams(use_tc_tiling_on_sc=False),
        out_type=[jax.ShapeDtypeStruct((T, ED), jnp.float32) for _ in range(5)],
        scratch_types=(
            [pltpu.VMEM((2, CH), jnp.int32) for _ in range(5)]
            + [pltpu.VMEM((2, CH, ED), jnp.float32) for _ in range(5)]
            + [pltpu.SemaphoreType.DMA, pltpu.SemaphoreType.DMA,
               pltpu.SemaphoreType.DMA]
        ),
    )
    def k(t0, t1, t2, t3, t4, i5,
          o0, o1, o2, o3, o4, v0, v1, v2, v3, v4,
          r0, r1, r2, r3, r4, sem_i, sem_g, sem_w):
        tabs = (t0, t1, t2, t3, t4)
        idxv = (v0, v1, v2, v3, v4)
        rows = (r0, r1, r2, r3, r4)
        outs = (o0, o1, o2, o3, o4)
        wid = lax.axis_index("s") * NC + lax.axis_index("c")
        base = wid * per_w

        def idx_cps(c, b, mk):
            return [mk(i5.at[f, pl.ds(base + c * CH, CH)], idxv[f].at[b],
                       sem_i) for f in range(5)]

        def gather_cps(b, mk):
            return [mk(tabs[f].at[idxv[f].at[b]], rows[f].at[b], sem_g)
                    for f in range(5)]

        def write_cps(c, b, mk):
            return [mk(rows[f].at[b], outs[f].at[pl.ds(base + c * CH, CH)],
                       sem_w) for f in range(5)]

        issue, mk = pltpu.async_copy, pltpu.make_async_copy

        idx_cps(0, 0, issue)
        for cp in idx_cps(0, 0, mk):
            cp.wait()
        gather_cps(0, issue)
        idx_cps(1, 1, issue)

        def half(c, b):
            for cp in gather_cps(b, mk):
                cp.wait()

            @pl.when(c >= 1)
            def _():
                for cp in write_cps(c - 1, 1 - b, mk):
                    cp.wait()

            @pl.when(c + 1 < n_ch)
            def _():
                for cp in idx_cps(c + 1, 1 - b, mk):
                    cp.wait()
                gather_cps(1 - b, issue)

            @pl.when(c + 2 < n_ch)
            def _():
                idx_cps(c + 2, b, issue)

            write_cps(c, b, issue)

        def body(c2, carry):
            half(2 * c2, 0)
            half(2 * c2 + 1, 1)
            return carry

        lax.fori_loop(0, n_ch // 2, body, 0)
        for cp in write_cps(n_ch - 1, 1, mk):
            cp.wait()

    return k(*tables, idx5)


def _tc_body(inter_r, el_r, td_r, ga, gt, gg, gu, gi,
             eint, wci, wca, wct, wcg, wcu, wcit, wea, wet, weg,
             wcel, wctd, weel, wetd, bc, be, eo, xo):
    f32 = jnp.float32
    fields = (ga, gt, gg, gu, gi)
    wx = (wca, wct, wcg, wcu, wcit)
    we = (wea, wet, weg)
    m3 = jnp.dot(eint[...], wci[...], preferred_element_type=f32)
    inter = inter_r[0, 0, :]
    elv = el_r[0, 0, :]
    tdv = td_r[0, 0, :]
    for p in range(PP):
        sl = slice(p * ED, (p + 1) * ED)
        x = jnp.dot(fields[0][...][:, sl], wx[0][...],
                    preferred_element_type=f32)
        for f in range(1, 5):
            x += jnp.dot(fields[f][...][:, sl], wx[f][...],
                         preferred_element_type=f32)
        e = jnp.dot(fields[0][...][:, sl], we[0][...],
                    preferred_element_type=f32)
        e += jnp.dot(fields[1][...][:, sl], we[1][...],
                     preferred_element_type=f32)
        e += jnp.dot(fields[2][...][:, sl], we[2][...],
                     preferred_element_type=f32)
        ii = inter[p * NBATCH:(p + 1) * NBATCH][:, None]
        x += jnp.where(ii == 0, 1.0, 0.0) * m3[0:1, :]
        x += jnp.where(ii == 1, 1.0, 0.0) * m3[1:2, :]
        x += jnp.where(ii == 2, 1.0, 0.0) * m3[2:3, :]
        el = elv[p * NBATCH:(p + 1) * NBATCH][:, None]
        td = tdv[p * NBATCH:(p + 1) * NBATCH][:, None]
        x += el * wcel[...]
        x += td * wctd[...]
        x += bc[...]
        e += el * weel[...]
        e += td * wetd[...]
        e += be[...]
        xo[p] = x.T
        eo[p] = e.T


def _tc_project(inter3, el3, td3, gs, eint, wblocks, T, S):
    NB = T // TB
    pair_spec = pl.BlockSpec((TB // 2, 2 * ED), lambda i: (i, 0))
    tok_spec = pl.BlockSpec((1, 1, TB), lambda i: (i, 0, 0))
    full = lambda s: pl.BlockSpec(s, lambda i: (0, 0))
    in_specs = (
        [tok_spec, tok_spec, tok_spec]
        + [pair_spec] * 5
        + [full(w.shape) for w in ([eint] + list(wblocks))]
    )
    out_specs = [pl.BlockSpec((PP, HD, NBATCH), lambda i: (i, 0, 0))] * 2
    out_shape = [jax.ShapeDtypeStruct((S, HD, NBATCH), jnp.float32)] * 2
    return pl.pallas_call(
        _tc_body,
        grid=(NB,),
        in_specs=in_specs,
        out_specs=out_specs,
        out_shape=out_shape,
    )(inter3, el3, td3, *gs, eint, *wblocks)


def kernel(interaction, user_idx, item_idx, assessmentItemID, testId, KnowledgeTag,
           elapsed, time_diff, user_emb, item_emb, emb_interaction, emb_assess,
           emb_test, emb_tag, W_comb, b_comb, W_enc, b_enc):
    B, S = interaction.shape
    T = B * S
    NB = T // TB

    i32 = jnp.int32
    idx5 = jnp.stack([assessmentItemID, testId, KnowledgeTag,
                      user_idx, item_idx]).astype(i32)
    idx5 = (idx5.reshape(5, NBATCH, NB, PP).transpose(0, 2, 1, 3)
            .reshape(5, T))

    gs = _sc_gather5(
        (emb_assess, emb_test, emb_tag, user_emb, item_emb), idx5, T)
    gs = [g.reshape(T // 2, 2 * ED) for g in gs]

    trio = jnp.stack([interaction.astype(i32),
                      lax.bitcast_convert_type(elapsed, i32),
                      lax.bitcast_convert_type(time_diff, i32)])
    trio = trio.transpose(0, 2, 1).reshape(3, NB, 1, TB)
    inter3 = trio[0]
    el3 = lax.bitcast_convert_type(trio[1], jnp.float32)
    td3 = lax.bitcast_convert_type(trio[2], jnp.float32)

    wblocks = (
        W_comb[0:64],
        W_comb[64:128],
        W_comb[128:192],
        W_comb[192:256],
        W_comb[258:322],
        W_comb[322:386],
        W_enc[0:64],
        W_enc[64:128],
        W_enc[128:192],
        W_comb[256:257],
        W_comb[257:258],
        W_enc[192:193],
        W_enc[193:194],
        b_comb.reshape(1, HD),
        b_enc.reshape(1, HD),
    )
    enc_x, x = _tc_project(inter3, el3, td3, gs, emb_interaction, wblocks, T, S)
    return (jnp.transpose(enc_x, (2, 0, 1)), jnp.transpose(x, (2, 0, 1)))


# speedup vs baseline: 4.7977x; 1.0050x over previous
import functools

import jax
import jax.numpy as jnp
from jax import lax
from jax.experimental import pallas as pl
from jax.experimental.pallas import tpu as pltpu
from jax.experimental.pallas import tpu_sc as plsc

HD = 192
ED = 64
NBATCH = 1024
PP = 2
TB = PP * NBATCH
SPOS = 4
CH = 128


def _sc_gather5(tables, idx5, T):
    info = plsc.get_sparse_core_info()
    NC, NS = info.num_cores, info.num_subcores
    NW = NC * NS
    per_w = T // NW
    n_ch = per_w // CH
    mesh = plsc.VectorSubcoreMesh(core_axis_name="c", subcore_axis_name="s")

    @functools.partial(
        pl.kernel,
        mesh=mesh,
        compiler_params=pltpu.CompilerParams(use_tc_tiling_on_sc=False),
        out_type=[jax.ShapeDtypeStruct((T, ED), jnp.float32) for _ in range(5)],
        scratch_types=(
            [pltpu.VMEM((2, CH), jnp.int32) for _ in range(5)]
            + [pltpu.VMEM((2, CH, ED), jnp.float32) for _ in range(5)]
            + [pltpu.SemaphoreType.DMA, pltpu.SemaphoreType.DMA,
               pltpu.SemaphoreType.DMA]
        ),
    )
    def k(t0, t1, t2, t3, t4, i5,
          o0, o1, o2, o3, o4, v0, v1, v2, v3, v4,
          r0, r1, r2, r3, r4, sem_i, sem_g, sem_w):
        tabs = (t0, t1, t2, t3, t4)
        idxv = (v0, v1, v2, v3, v4)
        rows = (r0, r1, r2, r3, r4)
        outs = (o0, o1, o2, o3, o4)
        wid = lax.axis_index("s") * NC + lax.axis_index("c")
        base = wid * per_w

        def idx_cps(c, b, mk):
            return [mk(i5.at[f, pl.ds(base + c * CH, CH)], idxv[f].at[b],
                       sem_i) for f in range(5)]

        def gather_cps(b, mk):
            return [mk(tabs[f].at[idxv[f].at[b]], rows[f].at[b], sem_g)
                    for f in range(5)]

        def write_cps(c, b, mk):
            return [mk(rows[f].at[b], outs[f].at[pl.ds(base + c * CH, CH)],
                       sem_w) for f in range(5)]

        issue, mk = pltpu.async_copy, pltpu.make_async_copy

        idx_cps(0, 0, issue)
        for cp in idx_cps(0, 0, mk):
            cp.wait()
        gather_cps(0, issue)
        idx_cps(1, 1, issue)

        def half(c, b):
            for cp in gather_cps(b, mk):
                cp.wait()

            @pl.when(c >= 1)
            def _():
                for cp in write_cps(c - 1, 1 - b, mk):
                    cp.wait()

            @pl.when(c + 1 < n_ch)
            def _():
                for cp in idx_cps(c + 1, 1 - b, mk):
                    cp.wait()
                gather_cps(1 - b, issue)

            @pl.when(c + 2 < n_ch)
            def _():
                idx_cps(c + 2, b, issue)

            write_cps(c, b, issue)

        def body(c2, carry):
            half(2 * c2, 0)
            half(2 * c2 + 1, 1)
            return carry

        lax.fori_loop(0, n_ch // 2, body, 0)
        for cp in write_cps(n_ch - 1, 1, mk):
            cp.wait()

    return k(*tables, idx5)


def _tc_body(inter_r, el_r, td_r, ga, gt, gg, gu, gi,
             eint, wci, wca, wct, wcg, wcu, wcit, wea, wet, weg,
             wcel, wctd, weel, wetd, bc, be, eo, xo):
    f32 = jnp.float32
    fields = (ga, gt, gg, gu, gi)
    wx = (wca, wct, wcg, wcu, wcit)
    we = (wea, wet, weg)
    m3 = jnp.dot(eint[...], wci[...], preferred_element_type=f32)
    for q in range(SPOS):
        rows = slice((q // 2) * NBATCH, (q // 2 + 1) * NBATCH)
        sl = slice((q % 2) * ED, (q % 2 + 1) * ED)
        x = jnp.dot(fields[0][rows, sl], wx[0][...],
                    preferred_element_type=f32)
        for f in range(1, 5):
            x += jnp.dot(fields[f][rows, sl], wx[f][...],
                         preferred_element_type=f32)
        e = jnp.dot(fields[0][rows, sl], we[0][...],
                    preferred_element_type=f32)
        e += jnp.dot(fields[1][rows, sl], we[1][...],
                     preferred_element_type=f32)
        e += jnp.dot(fields[2][rows, sl], we[2][...],
                     preferred_element_type=f32)
        ii = inter_r[0, q, :][:, None]
        x += jnp.where(ii == 0, 1.0, 0.0) * m3[0:1, :]
        x += jnp.where(ii == 1, 1.0, 0.0) * m3[1:2, :]
        x += jnp.where(ii == 2, 1.0, 0.0) * m3[2:3, :]
        el = el_r[0, q, :][:, None]
        td = td_r[0, q, :][:, None]
        x += el * wcel[...]
        x += td * wctd[...]
        x += bc[...]
        e += el * weel[...]
        e += td * wetd[...]
        e += be[...]
        xo[q] = x.T
        eo[q] = e.T


def _tc_project(inter_t, el_t, td_t, gs, eint, wblocks, T, S):
    NK = S // SPOS
    pair_spec = pl.BlockSpec((SPOS * NBATCH // 2, 2 * ED), lambda i: (i, 0))
    tok_spec = pl.BlockSpec((1, SPOS, NBATCH), lambda i: (i, 0, 0))
    full = lambda s: pl.BlockSpec(s, lambda i: (0, 0))
    in_specs = (
        [tok_spec, tok_spec, tok_spec]
        + [pair_spec] * 5
        + [full(w.shape) for w in ([eint] + list(wblocks))]
    )
    out_specs = [pl.BlockSpec((SPOS, HD, NBATCH), lambda i: (i, 0, 0))] * 2
    out_shape = [jax.ShapeDtypeStruct((S, HD, NBATCH), jnp.float32)] * 2
    return pl.pallas_call(
        _tc_body,
        grid=(NK,),
        in_specs=in_specs,
        out_specs=out_specs,
        out_shape=out_shape,
    )(inter_t, el_t, td_t, *gs, eint, *wblocks)


def kernel(interaction, user_idx, item_idx, assessmentItemID, testId, KnowledgeTag,
           elapsed, time_diff, user_emb, item_emb, emb_interaction, emb_assess,
           emb_test, emb_tag, W_comb, b_comb, W_enc, b_enc):
    B, S = interaction.shape
    T = B * S
    NB = T // TB

    i32 = jnp.int32
    idx5 = jnp.stack([assessmentItemID, testId, KnowledgeTag,
                      user_idx, item_idx]).astype(i32)
    idx5 = (idx5.reshape(5, NBATCH, NB, PP).transpose(0, 2, 1, 3)
            .reshape(5, T))

    gs = _sc_gather5(
        (emb_assess, emb_test, emb_tag, user_emb, item_emb), idx5, T)
    gs = [g.reshape(T // 2, 2 * ED) for g in gs]

    nk = S // SPOS
    inter_t = interaction.T.astype(i32).reshape(nk, SPOS, NBATCH)
    el_t = elapsed.T.reshape(nk, SPOS, NBATCH)
    td_t = time_diff.T.reshape(nk, SPOS, NBATCH)

    wblocks = (
        W_comb[0:64],
        W_comb[64:128],
        W_comb[128:192],
        W_comb[192:256],
        W_comb[258:322],
        W_comb[322:386],
        W_enc[0:64],
        W_enc[64:128],
        W_enc[128:192],
        W_comb[256:257],
        W_comb[257:258],
        W_enc[192:193],
        W_enc[193:194],
        b_comb.reshape(1, HD),
        b_enc.reshape(1, HD),
    )
    enc_x, x = _tc_project(inter_t, el_t, td_t, gs, emb_interaction, wblocks,
                           T, S)
    return (jnp.transpose(enc_x, (2, 0, 1)), jnp.transpose(x, (2, 0, 1)))
